# baseline (speedup 1.0000x reference)
import functools

import jax
import jax.numpy as jnp
from jax import lax
from jax.experimental import pallas as pl
from jax.experimental.pallas import tpu as pltpu
from jax.experimental.pallas import tpu_sc as plsc

_SEQ = 2048
_EMBED = 768
_BATCH = 4
_NC = 2
_NS = 16
_NW = _NC * _NS
_S_PER_W = _SEQ // _NW
_CHUNK = 32
_NCH = _S_PER_W // _CHUNK
_NTOT = _BATCH * _NCH
_NBUF = 4
_LANES = 16
_VREGS = _EMBED // _LANES


def _body(ids_hbm, wte_hbm, wpe_hbm, out_hbm, idx_v, pos_v, bufs, gsems,
          osems, psems, isem):
    cid = lax.axis_index("c")
    sid = lax.axis_index("s")
    wid = sid * _NC + cid
    s0 = wid * _S_PER_W

    id_copies = [
        pltpu.async_copy(ids_hbm.at[pl.ds(b * _SEQ + s0, _S_PER_W)],
                         idx_v.at[b], isem)
        for b in range(_BATCH)
    ]
    pos_copies = [None] * _NCH
    pos_copies[0] = pltpu.async_copy(wpe_hbm.at[pl.ds(s0, _CHUNK)], pos_v,
                                     psems[0])
    for c in id_copies:
        c.wait()

    def fire_gather(k):
        c, b = divmod(k, _BATCH)
        i = k % _NBUF
        idx = idx_v.at[b, pl.ds(c * _CHUNK, _CHUNK)]
        return pltpu.async_copy(wte_hbm.at[idx], bufs[i], gsems[i])

    def fire_out(k):
        c, b = divmod(k, _BATCH)
        i = k % _NBUF
        dst = out_hbm.at[pl.ds(b * _SEQ + s0 + c * _CHUNK, _CHUNK)]
        return pltpu.async_copy(bufs[i], dst, osems[i])

    gcopies = [None] * _NTOT
    ocopies = [None] * _NTOT
    for k in range(_NBUF):
        gcopies[k] = fire_gather(k)
    pos_copies[0].wait()

    for p in range(_NTOT // 2):
        k = 2 * p
        c, b = divmod(k, _BATCH)
        gcopies[k].wait()
        gcopies[k + 1].wait()
        if b == 0 and c > 0:
            pos_copies[c].wait()
        buf_a = bufs[k % _NBUF]
        buf_b = bufs[(k + 1) % _NBUF]

        @plsc.parallel_loop(0, _CHUNK, 1, unroll=4)
        def row_body(r, buf_a=buf_a, buf_b=buf_b):
            for j in range(_VREGS):
                sl = pl.ds(j * _LANES, _LANES)
                v = pos_v[r, sl]
                plsc.addupdate(buf_a.at[r, sl], v)
                plsc.addupdate(buf_b.at[r, sl], v)

        if b + 1 == _BATCH - 1 and c + 1 < _NCH:
            pos_copies[c + 1] = pltpu.async_copy(
                wpe_hbm.at[pl.ds(s0 + (c + 1) * _CHUNK, _CHUNK)], pos_v,
                psems[c + 1])
        ocopies[k] = fire_out(k)
        ocopies[k + 1] = fire_out(k + 1)
        if k + _NBUF < _NTOT:
            ocopies[k].wait()
            ocopies[k + 1].wait()
            gcopies[k + _NBUF] = fire_gather(k + _NBUF)
            gcopies[k + _NBUF + 1] = fire_gather(k + _NBUF + 1)

    for k in range(_NTOT - _NBUF, _NTOT):
        ocopies[k].wait()


_emb = functools.partial(
    pl.kernel,
    out_type=jax.ShapeDtypeStruct((_BATCH * _SEQ, _EMBED), jnp.float32),
    mesh=plsc.VectorSubcoreMesh(core_axis_name="c", subcore_axis_name="s"),
    scratch_types=[
        pltpu.VMEM((_BATCH, _S_PER_W), jnp.int32),
        pltpu.VMEM((_CHUNK, _EMBED), jnp.float32),
        [pltpu.VMEM((_CHUNK, _EMBED), jnp.float32) for _ in range(_NBUF)],
        [pltpu.SemaphoreType.DMA for _ in range(_NBUF)],
        [pltpu.SemaphoreType.DMA for _ in range(_NBUF)],
        [pltpu.SemaphoreType.DMA for _ in range(_NCH)],
        pltpu.SemaphoreType.DMA,
    ],
)(_body)


@jax.jit
def kernel(input_ids, token_embeddings, position_embeddings):
    ids = input_ids.reshape(-1).astype(jnp.int32)
    out = _emb(ids, token_embeddings, position_embeddings)
    return out.reshape(_BATCH, _SEQ, _EMBED)


# speedup vs baseline: 1.1405x; 1.1405x over previous
import functools

import jax
import jax.numpy as jnp
from jax import lax
from jax.experimental import pallas as pl
from jax.experimental.pallas import tpu as pltpu
from jax.experimental.pallas import tpu_sc as plsc

_SEQ = 2048
_EMBED = 768
_BATCH = 4
_NC = 2
_NS = 16
_NW = _NC * _NS
_S_PER_W = _SEQ // _NW
_CHUNK = 32
_NCH = _S_PER_W // _CHUNK
_NTOT = _BATCH * _NCH
_NBUF = 4
_LANES = 16
_VREGS = _EMBED // _LANES


def _body(ids_hbm, wte_hbm, wpe_hbm, out_hbm, idx_v, pos_v, bufs, gsems,
          osems, psems, isem):
    cid = lax.axis_index("c")
    sid = lax.axis_index("s")
    wid = sid * _NC + cid
    s0 = wid * _S_PER_W

    id_copies = [
        pltpu.async_copy(ids_hbm.at[pl.ds(b * _SEQ + s0, _S_PER_W)],
                         idx_v.at[b], isem)
        for b in range(_BATCH)
    ]
    pos_copies = [None] * _NCH
    pos_copies[0] = pltpu.async_copy(wpe_hbm.at[pl.ds(s0, _CHUNK)], pos_v,
                                     psems[0])
    for c in id_copies:
        c.wait()

    def fire_gather(k):
        c, b = divmod(k, _BATCH)
        i = k % _NBUF
        idx = idx_v.at[b, pl.ds(c * _CHUNK, _CHUNK)]
        return pltpu.async_copy(wte_hbm.at[idx], bufs[i], gsems[i])

    def fire_out(k):
        c, b = divmod(k, _BATCH)
        i = k % _NBUF
        dst = out_hbm.at[pl.ds(b * _SEQ + s0 + c * _CHUNK, _CHUNK)]
        return pltpu.async_copy(bufs[i], dst, osems[i])

    gcopies = [None] * _NTOT
    ocopies = [None] * _NTOT
    for k in range(_NBUF):
        gcopies[k] = fire_gather(k)
    pos_copies[0].wait()

    for p in range(_NTOT // 2):
        k = 2 * p
        c, b = divmod(k, _BATCH)
        gcopies[k].wait()
        gcopies[k + 1].wait()
        if b == 0 and c > 0:
            pos_copies[c].wait()
        buf_a = bufs[k % _NBUF]
        buf_b = bufs[(k + 1) % _NBUF]

        @plsc.parallel_loop(0, _CHUNK, 1, unroll=1)
        def row_body(r, buf_a=buf_a, buf_b=buf_b):
            for j in range(_VREGS):
                sl = pl.ds(j * _LANES, _LANES)
                v = pos_v[r, sl]
                plsc.addupdate(buf_a.at[r, sl], v)
                plsc.addupdate(buf_b.at[r, sl], v)

        if b + 1 == _BATCH - 1 and c + 1 < _NCH:
            pos_copies[c + 1] = pltpu.async_copy(
                wpe_hbm.at[pl.ds(s0 + (c + 1) * _CHUNK, _CHUNK)], pos_v,
                psems[c + 1])
        ocopies[k] = fire_out(k)
        ocopies[k + 1] = fire_out(k + 1)
        if k + _NBUF < _NTOT:
            ocopies[k].wait()
            ocopies[k + 1].wait()
            gcopies[k + _NBUF] = fire_gather(k + _NBUF)
            gcopies[k + _NBUF + 1] = fire_gather(k + _NBUF + 1)

    for k in range(_NTOT - _NBUF, _NTOT):
        ocopies[k].wait()


_emb = functools.partial(
    pl.kernel,
    out_type=jax.ShapeDtypeStruct((_BATCH * _SEQ, _EMBED), jnp.float32),
    mesh=plsc.VectorSubcoreMesh(core_axis_name="c", subcore_axis_name="s"),
    scratch_types=[
        pltpu.VMEM((_BATCH, _S_PER_W), jnp.int32),
        pltpu.VMEM((_CHUNK, _EMBED), jnp.float32),
        [pltpu.VMEM((_CHUNK, _EMBED), jnp.float32) for _ in range(_NBUF)],
        [pltpu.SemaphoreType.DMA for _ in range(_NBUF)],
        [pltpu.SemaphoreType.DMA for _ in range(_NBUF)],
        [pltpu.SemaphoreType.DMA for _ in range(_NCH)],
        pltpu.SemaphoreType.DMA,
    ],
)(_body)


@jax.jit
def kernel(input_ids, token_embeddings, position_embeddings):
    ids = input_ids.reshape(-1).astype(jnp.int32)
    out = _emb(ids, token_embeddings, position_embeddings)
    return out.reshape(_BATCH, _SEQ, _EMBED)
